# baseline (speedup 1.0000x reference)
import functools

import jax
import jax.numpy as jnp
from jax import lax
from jax.experimental import pallas as pl
from jax.experimental.pallas import tpu as pltpu
from jax.experimental.pallas import tpu_sc as plsc

_LAMBDA = 0.001
_SIZE = 1000000
_BATCH = 16384
_FACTOR = 16

_info = plsc.get_sparse_core_info()
_NC, _NS, _L = _info.num_cores, _info.num_subcores, _info.num_lanes
_NW = _NC * _NS
_BPW = _BATCH // _NW
_GR = 15
_NGRP = 36
_IDXPAD = 640
_TAIL = (_SIZE // 128) * 128
_LASTB = _TAIL // 128 - 1


@functools.partial(
    pl.kernel,
    out_type=jax.ShapeDtypeStruct((_NW, 3 * _L), jnp.float32),
    mesh=plsc.VectorSubcoreMesh(core_axis_name="c", subcore_axis_name="s"),
    compiler_params=pltpu.CompilerParams(needs_layout_passes=False),
    scratch_types=[
        pltpu.VMEM((_IDXPAD,), jnp.int32),
        pltpu.VMEM((_IDXPAD,), jnp.int32),
        pltpu.VMEM((_IDXPAD,), jnp.float32),
        pltpu.VMEM((_GR, _FACTOR, 128), jnp.float32),
        pltpu.VMEM((_GR, _FACTOR, 128), jnp.float32),
        pltpu.VMEM((_GR, _FACTOR, 128), jnp.float32),
        pltpu.VMEM((_GR, _FACTOR, 128), jnp.float32),
        pltpu.VMEM((_FACTOR, 128), jnp.float32),
        pltpu.VMEM((_FACTOR, 128), jnp.float32),
        pltpu.VMEM((3 * _L,), jnp.float32),
        pltpu.SemaphoreType.DMA,
        pltpu.SemaphoreType.DMA,
        pltpu.SemaphoreType.DMA,
        pltpu.SemaphoreType.DMA,
    ],
)
def _bpr_partials(uidx_hbm, iidx_hbm, rat_hbm, wu_hbm, wi_hbm,
                  wu_tail_hbm, wi_tail_hbm, out_hbm,
                  uidx_v, iidx_v, rat_v, ubuf_a, ibuf_a, ubuf_b, ibuf_b,
                  utail_v, itail_v, out_v, sem_ua, sem_ia, sem_ub, sem_ib):
    wid = lax.axis_index("s") * _NC + lax.axis_index("c")
    base = wid * _BPW
    pltpu.sync_copy(uidx_hbm.at[pl.ds(base, _BPW)],
                    uidx_v.at[pl.ds(0, _BPW)])
    pltpu.sync_copy(iidx_hbm.at[pl.ds(base, _BPW)],
                    iidx_v.at[pl.ds(0, _BPW)])
    pltpu.sync_copy(rat_hbm.at[pl.ds(base, _BPW)], rat_v.at[pl.ds(0, _BPW)])
    pltpu.sync_copy(wu_tail_hbm, utail_v)
    pltpu.sync_copy(wi_tail_hbm, itail_v)

    lanes = lax.iota(jnp.int32, _L)
    zero = jnp.zeros((_L,), jnp.float32)
    blk_lim = jnp.full((_L,), _LASTB, jnp.int32)
    lane_lim = jnp.full((_L,), 127, jnp.int32)

    def blocks(g):
        uvec = uidx_v[pl.ds(g * _GR, _L)]
        ivec = iidx_v[pl.ds(g * _GR, _L)]
        ub = jnp.clip(uvec >> 7, 0, blk_lim)
        ib = jnp.clip(ivec >> 7, 0, blk_lim)
        return uvec, ivec, ub, ib

    def issue(g, ubuf, ibuf, sem_u, sem_i):
        _, _, ub, ib = blocks(g)
        for k in range(_GR):
            us = pl.multiple_of(ub[k] * 128, 128)
            its = pl.multiple_of(ib[k] * 128, 128)
            pltpu.async_copy(wu_hbm.at[:, pl.ds(us, 128)], ubuf.at[k], sem_u)
            pltpu.async_copy(wi_hbm.at[:, pl.ds(its, 128)], ibuf.at[k], sem_i)

    def drain(ubuf, ibuf, sem_u, sem_i):
        for k in range(_GR):
            pltpu.make_async_copy(
                wu_hbm.at[:, pl.ds(0, 128)], ubuf.at[k], sem_u).wait()
            pltpu.make_async_copy(
                wi_hbm.at[:, pl.ds(0, 128)], ibuf.at[k], sem_i).wait()

    def compute(g, ubuf, ibuf, carry):
        task_acc, usq_acc, isq_acc = carry
        uvec, ivec, ub, ib = blocks(g)
        valid = (lanes < _GR) & (g * _GR + lanes < _BPW)
        ulane = jnp.clip(uvec - ub * 128, 0, lane_lim)
        ilane = jnp.clip(ivec - ib * 128, 0, lane_lim)
        utail_m = (uvec >= _TAIL) & valid
        itail_m = (ivec >= _TAIL) & valid
        ulane_t = jnp.clip(uvec - _TAIL, 0, lane_lim)
        ilane_t = jnp.clip(ivec - _TAIL, 0, lane_lim)
        pred = zero
        for d in range(_FACTOR):
            dsplat = jnp.full((_L,), d, jnp.int32)
            u_m = plsc.load_gather(ubuf, [jnp.minimum(lanes, _GR - 1), dsplat, ulane])
            i_m = plsc.load_gather(ibuf, [jnp.minimum(lanes, _GR - 1), dsplat, ilane])
            u_t = plsc.load_gather(utail_v, [dsplat, ulane_t], mask=utail_m)
            i_t = plsc.load_gather(itail_v, [dsplat, ilane_t], mask=itail_m)
            u = jnp.where(utail_m, u_t, u_m)
            it = jnp.where(itail_m, i_t, i_m)
            u = jnp.where(valid, u, zero)
            it = jnp.where(valid, it, zero)
            pred = pred + u * it
            usq_acc = usq_acc + u * u
            isq_acc = isq_acc + it * it
        diff = pred - rat_v[pl.ds(g * _GR, _L)]
        task_acc = task_acc + jnp.where(valid, diff * diff, zero)
        return task_acc, usq_acc, isq_acc

    last_g = jnp.int32(_NGRP - 1)

    def step(j, carry):
        g_a = 2 * j
        g_b = 2 * j + 1
        issue(g_b, ubuf_b, ibuf_b, sem_ub, sem_ib)
        drain(ubuf_a, ibuf_a, sem_ua, sem_ia)
        carry = compute(g_a, ubuf_a, ibuf_a, carry)
        issue(jnp.minimum(g_a + 2, last_g), ubuf_a, ibuf_a, sem_ua, sem_ia)
        drain(ubuf_b, ibuf_b, sem_ub, sem_ib)
        return compute(g_b, ubuf_b, ibuf_b, carry)

    issue(0, ubuf_a, ibuf_a, sem_ua, sem_ia)
    carry = lax.fori_loop(0, _NGRP // 2, step, (zero, zero, zero))
    drain(ubuf_a, ibuf_a, sem_ua, sem_ia)

    task_acc, usq_acc, isq_acc = carry
    out_v[pl.ds(0, _L)] = task_acc
    out_v[pl.ds(_L, _L)] = usq_acc
    out_v[pl.ds(2 * _L, _L)] = isq_acc
    pltpu.sync_copy(out_v, out_hbm.at[wid])


def kernel(user0, item_i0, ratings, W_user, W_item):
    rat = ratings.astype(jnp.float32)
    wu_tail = jnp.zeros((_FACTOR, 128), jnp.float32).at[:, :_SIZE - _TAIL].set(
        W_user[_TAIL:].T)
    wi_tail = jnp.zeros((_FACTOR, 128), jnp.float32).at[:, :_SIZE - _TAIL].set(
        W_item[_TAIL:].T)
    partials = _bpr_partials(user0, item_i0, rat, W_user.T, W_item.T,
                             wu_tail, wi_tail)
    p = partials.reshape(_NW, 3, _L)
    task_loss = p[:, 0, :].sum() / _BATCH
    l2 = _LAMBDA * (p[:, 1, :].sum() + p[:, 2, :].sum()) / (_BATCH * _FACTOR)
    loss = task_loss + l2
    return (loss, task_loss, l2)
